# baseline (speedup 1.0000x reference)
import functools

import jax
import jax.numpy as jnp
from jax import lax
from jax.experimental import pallas as pl
from jax.experimental.pallas import tpu as pltpu
from jax.experimental.pallas import tpu_sc as plsc

N_NODES = 10000
E_TOTAL = 320000
DIM = 128
HALF = DIM // 2

NC = 2
NS = 16
NW = NC * NS
CHUNK = 80
E_PER_W = E_TOTAL // NW
N_CHUNK_G = E_PER_W // CHUNK
N_PAIR_G = N_CHUNK_G // 2
E_PER_S = E_TOTAL // NS
N_CHUNK_S = E_PER_S // CHUNK
N_PAIR_S = N_CHUNK_S // 2
ROWS_PER_TILE = N_NODES // NS
ACC_ROWS = N_NODES


def _mm_body(xs_ref, xd_ref, ws_ref, wd_ref, b_ref, wself_ref, wmsg_ref,
             hst_ref, hwt_ref):
    hp = (jnp.dot(xs_ref[...], ws_ref[...], preferred_element_type=jnp.float32)
          + jnp.dot(xd_ref[...], wd_ref[...], preferred_element_type=jnp.float32)
          + b_ref[...])
    hs = jnp.dot(hp, wself_ref[...], preferred_element_type=jnp.float32)
    hst_ref[...] = jnp.stack([hs[:, :HALF], hs[:, HALF:]], axis=0)
    hw = jnp.dot(hp, wmsg_ref[...], preferred_element_type=jnp.float32)
    hwt_ref[...] = jnp.stack([hw[:, :HALF], hw[:, HALF:]], axis=0)


def _seg_body(hwt_hbm, hst_hbm, src_hbm, dst_hbm, ht_hbm,
              sidx, didx, rows_a, rows_b, table, acc,
              g_a, g_b, sc_a, sc_b):
    c = lax.axis_index("c")
    s = lax.axis_index("s")
    stripe = s * ROWS_PER_TILE
    pltpu.sync_copy(hst_hbm.at[c].at[pl.ds(stripe, ROWS_PER_TILE)],
                    acc.at[pl.ds(stripe, ROWS_PER_TILE)])
    pltpu.sync_copy(hwt_hbm.at[c].at[pl.ds(stripe, ROWS_PER_TILE)],
                    table.at[pl.ds(stripe, ROWS_PER_TILE)])
    pltpu.sync_copy(src_hbm.at[s], sidx)
    pltpu.sync_copy(dst_hbm.at[s], didx)
    plsc.subcore_barrier()

    pltpu.async_copy(table.at[sidx.at[0]], rows_a, g_a)

    def pair(i, carry):
        a, b = 2 * i, 2 * i + 1

        @pl.when(i >= 1)
        def _():
            pltpu.make_async_copy(rows_b, acc.at[didx.at[0]], sc_b).wait()

        gb = pltpu.async_copy(table.at[sidx.at[b]], rows_b, g_b)
        pltpu.make_async_copy(table.at[sidx.at[a]], rows_a, g_a).wait()
        sca = pltpu.async_copy(rows_a, acc.at[didx.at[a]], sc_a, add=True)
        sca.wait()

        @pl.when(i < N_PAIR_S - 1)
        def _():
            pltpu.async_copy(table.at[sidx.at[a + 2]], rows_a, g_a)

        gb.wait()
        pltpu.async_copy(rows_b, acc.at[didx.at[b]], sc_b, add=True)
        return carry

    lax.fori_loop(0, N_PAIR_S, pair, 0)
    pltpu.make_async_copy(rows_b, acc.at[didx.at[0]], sc_b).wait()

    plsc.subcore_barrier()
    pltpu.sync_copy(acc.at[pl.ds(stripe, ROWS_PER_TILE)],
                    ht_hbm.at[c].at[pl.ds(stripe, ROWS_PER_TILE)])


def _gather_body(ht_hbm, src_hbm, dst_hbm, hsrc_hbm, hdst_hbm,
                 sidx, didx, srows_a, srows_b, drows_a, drows_b, table,
                 g_sa, g_sb, g_da, g_db, st_sa, st_sb, st_da, st_db):
    c = lax.axis_index("c")
    s = lax.axis_index("s")
    base = s * E_PER_S
    col = c * HALF
    stripe = s * ROWS_PER_TILE
    pltpu.sync_copy(ht_hbm.at[c].at[pl.ds(stripe, ROWS_PER_TILE)],
                    table.at[pl.ds(stripe, ROWS_PER_TILE)])
    pltpu.sync_copy(src_hbm.at[s], sidx)
    pltpu.sync_copy(dst_hbm.at[s], didx)
    plsc.subcore_barrier()
    pltpu.async_copy(table.at[sidx.at[0]], srows_a, g_sa)
    pltpu.async_copy(table.at[didx.at[0]], drows_a, g_da)

    def pair(i, carry):
        a, b = 2 * i, 2 * i + 1
        off_a = base + a * CHUNK
        off_b = base + b * CHUNK

        @pl.when(i >= 1)
        def _():
            pltpu.make_async_copy(
                srows_b, hsrc_hbm.at[pl.ds(off_b, CHUNK), pl.ds(col, HALF)],
                st_sb).wait()
            pltpu.make_async_copy(
                drows_b, hdst_hbm.at[pl.ds(off_b, CHUNK), pl.ds(col, HALF)],
                st_db).wait()

        gsb = pltpu.async_copy(table.at[sidx.at[b]], srows_b, g_sb)
        gdb = pltpu.async_copy(table.at[didx.at[b]], drows_b, g_db)

        pltpu.make_async_copy(table.at[sidx.at[a]], srows_a, g_sa).wait()
        ssa = pltpu.async_copy(
            srows_a, hsrc_hbm.at[pl.ds(off_a, CHUNK), pl.ds(col, HALF)], st_sa)
        pltpu.make_async_copy(table.at[didx.at[a]], drows_a, g_da).wait()
        sda = pltpu.async_copy(
            drows_a, hdst_hbm.at[pl.ds(off_a, CHUNK), pl.ds(col, HALF)], st_da)

        ssa.wait()

        @pl.when(i < N_PAIR_S - 1)
        def _():
            pltpu.async_copy(table.at[sidx.at[a + 2]], srows_a, g_sa)

        sda.wait()

        @pl.when(i < N_PAIR_S - 1)
        def _():
            pltpu.async_copy(table.at[didx.at[a + 2]], drows_a, g_da)

        gsb.wait()
        pltpu.async_copy(srows_b, hsrc_hbm.at[pl.ds(off_b, CHUNK),
                                              pl.ds(col, HALF)], st_sb)
        gdb.wait()
        pltpu.async_copy(drows_b, hdst_hbm.at[pl.ds(off_b, CHUNK),
                                              pl.ds(col, HALF)], st_db)
        return carry

    lax.fori_loop(0, N_PAIR_S, pair, 0)
    off_b = base + (N_CHUNK_S - 1) * CHUNK
    pltpu.make_async_copy(
        srows_b, hsrc_hbm.at[pl.ds(off_b, CHUNK), pl.ds(col, HALF)],
        st_sb).wait()
    pltpu.make_async_copy(
        drows_b, hdst_hbm.at[pl.ds(off_b, CHUNK), pl.ds(col, HALF)],
        st_db).wait()


@functools.lru_cache(maxsize=1)
def _sc_kernels():
    mesh = plsc.VectorSubcoreMesh(
        core_axis_name="c", subcore_axis_name="s",
        num_cores=NC, num_subcores=NS)
    seg = functools.partial(
        pl.kernel,
        out_type=jax.ShapeDtypeStruct((NC, N_NODES, HALF), jnp.float32),
        mesh=mesh,
        compiler_params=pltpu.CompilerParams(use_tc_tiling_on_sc=False),
        scratch_types=[
            pltpu.VMEM((N_CHUNK_S, CHUNK), jnp.int32),
            pltpu.VMEM((N_CHUNK_S, CHUNK), jnp.int32),
            pltpu.VMEM((CHUNK, HALF), jnp.float32),
            pltpu.VMEM((CHUNK, HALF), jnp.float32),
            pltpu.VMEM_SHARED((ACC_ROWS, HALF), jnp.float32),
            pltpu.VMEM_SHARED((ACC_ROWS, HALF), jnp.float32),
            pltpu.SemaphoreType.DMA,
            pltpu.SemaphoreType.DMA,
            pltpu.SemaphoreType.DMA,
            pltpu.SemaphoreType.DMA,
        ],
    )(_seg_body)
    gather = functools.partial(
        pl.kernel,
        out_type=[
            jax.ShapeDtypeStruct((E_TOTAL, DIM), jnp.float32),
            jax.ShapeDtypeStruct((E_TOTAL, DIM), jnp.float32),
        ],
        mesh=mesh,
        compiler_params=pltpu.CompilerParams(use_tc_tiling_on_sc=False),
        scratch_types=[
            pltpu.VMEM((N_CHUNK_S, CHUNK), jnp.int32),
            pltpu.VMEM((N_CHUNK_S, CHUNK), jnp.int32),
            pltpu.VMEM((CHUNK, HALF), jnp.float32),
            pltpu.VMEM((CHUNK, HALF), jnp.float32),
            pltpu.VMEM((CHUNK, HALF), jnp.float32),
            pltpu.VMEM((CHUNK, HALF), jnp.float32),
            pltpu.VMEM_SHARED((N_NODES, HALF), jnp.float32),
            pltpu.SemaphoreType.DMA,
            pltpu.SemaphoreType.DMA,
            pltpu.SemaphoreType.DMA,
            pltpu.SemaphoreType.DMA,
            pltpu.SemaphoreType.DMA,
            pltpu.SemaphoreType.DMA,
            pltpu.SemaphoreType.DMA,
            pltpu.SemaphoreType.DMA,
        ],
    )(_gather_body)
    return seg, gather


_ROW_BLK = 1000
_N_BLK = N_NODES // _ROW_BLK


def _relu_body(pre_ref, ht_ref):
    ht_ref[...] = jnp.maximum(pre_ref[...], 0.0)


_relu_call = pl.pallas_call(
    _relu_body,
    grid=(_N_BLK,),
    in_specs=[pl.BlockSpec((NC, _ROW_BLK, HALF), lambda i: (0, i, 0))],
    out_specs=pl.BlockSpec((NC, _ROW_BLK, HALF), lambda i: (0, i, 0)),
    out_shape=jax.ShapeDtypeStruct((NC, N_NODES, HALF), jnp.float32),
)

_mm_call = pl.pallas_call(
    _mm_body,
    grid=(_N_BLK,),
    in_specs=[
        pl.BlockSpec((_ROW_BLK, DIM), lambda i: (i, 0)),
        pl.BlockSpec((_ROW_BLK, DIM), lambda i: (i, 0)),
        pl.BlockSpec((DIM, DIM), lambda i: (0, 0)),
        pl.BlockSpec((DIM, DIM), lambda i: (0, 0)),
        pl.BlockSpec((1, DIM), lambda i: (0, 0)),
        pl.BlockSpec((DIM, DIM), lambda i: (0, 0)),
        pl.BlockSpec((DIM, DIM), lambda i: (0, 0)),
    ],
    out_specs=[
        pl.BlockSpec((NC, _ROW_BLK, HALF), lambda i: (0, i, 0)),
        pl.BlockSpec((NC, _ROW_BLK, HALF), lambda i: (0, i, 0)),
    ],
    out_shape=[
        jax.ShapeDtypeStruct((NC, N_NODES, HALF), jnp.float32),
        jax.ShapeDtypeStruct((NC, N_NODES, HALF), jnp.float32),
    ],
)


def kernel(edge_index, t, msg, x_src, x_dst, W_src, b_src, W_dst, b_dst,
           W_self, W_msg):
    del t, msg
    src3s = edge_index[0].reshape(NS, N_CHUNK_S, CHUNK)
    dst3s = edge_index[1].reshape(NS, N_CHUNK_S, CHUNK)
    bias = (b_src + b_dst).reshape(1, DIM)
    seg_kernel, gather_kernel = _sc_kernels()
    hst, hwt = _mm_call(x_src, x_dst, W_src, W_dst, bias, W_self, W_msg)
    pre = seg_kernel(hwt, hst, src3s, dst3s)
    ht = _relu_call(pre)
    h_src, h_dst = gather_kernel(ht, src3s, dst3s)
    return (h_src, h_dst)


# speedup vs baseline: 1.0690x; 1.0690x over previous
import functools

import jax
import jax.numpy as jnp
from jax import lax
from jax.experimental import pallas as pl
from jax.experimental.pallas import tpu as pltpu
from jax.experimental.pallas import tpu_sc as plsc

N_NODES = 10000
E_TOTAL = 320000
DIM = 128
HALF = DIM // 2

NC = 2
NS = 16
NW = NC * NS
CHUNK = 80
E_PER_W = E_TOTAL // NW
N_CHUNK_G = E_PER_W // CHUNK
N_PAIR_G = N_CHUNK_G // 2
E_PER_S = E_TOTAL // NS
N_CHUNK_S = E_PER_S // CHUNK
N_PAIR_S = N_CHUNK_S // 2
ROWS_PER_TILE = N_NODES // NS
ACC_ROWS = N_NODES


def _mm_body(xs_ref, xd_ref, ws_ref, wd_ref, b_ref, wself_ref, wmsg_ref,
             hst_ref, hwt_ref):
    hp = (jnp.dot(xs_ref[...], ws_ref[...], preferred_element_type=jnp.float32)
          + jnp.dot(xd_ref[...], wd_ref[...], preferred_element_type=jnp.float32)
          + b_ref[...])
    hs = jnp.dot(hp, wself_ref[...], preferred_element_type=jnp.float32)
    hst_ref[...] = jnp.stack([hs[:, :HALF], hs[:, HALF:]], axis=0)
    hw = jnp.dot(hp, wmsg_ref[...], preferred_element_type=jnp.float32)
    hwt_ref[...] = jnp.stack([hw[:, :HALF], hw[:, HALF:]], axis=0)


def _seg_body(hwt_hbm, hst_hbm, src_hbm, dst_hbm, ht_hbm,
              sidx, didx, rows_a, rows_b, table, acc,
              g_a, g_b, sc_a, sc_b):
    c = lax.axis_index("c")
    s = lax.axis_index("s")
    stripe = s * ROWS_PER_TILE
    pltpu.sync_copy(hst_hbm.at[c].at[pl.ds(stripe, ROWS_PER_TILE)],
                    acc.at[pl.ds(stripe, ROWS_PER_TILE)])
    pltpu.sync_copy(hwt_hbm.at[c].at[pl.ds(stripe, ROWS_PER_TILE)],
                    table.at[pl.ds(stripe, ROWS_PER_TILE)])
    pltpu.sync_copy(src_hbm.at[s], sidx)
    pltpu.sync_copy(dst_hbm.at[s], didx)
    plsc.subcore_barrier()

    pltpu.async_copy(table.at[sidx.at[0]], rows_a, g_a)

    def pair(i, carry):
        a, b = 2 * i, 2 * i + 1

        @pl.when(i >= 1)
        def _():
            pltpu.make_async_copy(rows_b, acc.at[didx.at[0]], sc_b).wait()

        gb = pltpu.async_copy(table.at[sidx.at[b]], rows_b, g_b)
        pltpu.make_async_copy(table.at[sidx.at[a]], rows_a, g_a).wait()
        sca = pltpu.async_copy(rows_a, acc.at[didx.at[a]], sc_a, add=True)
        sca.wait()

        @pl.when(i < N_PAIR_S - 1)
        def _():
            pltpu.async_copy(table.at[sidx.at[a + 2]], rows_a, g_a)

        gb.wait()
        pltpu.async_copy(rows_b, acc.at[didx.at[b]], sc_b, add=True)
        return carry

    lax.fori_loop(0, N_PAIR_S, pair, 0)
    pltpu.make_async_copy(rows_b, acc.at[didx.at[0]], sc_b).wait()

    plsc.subcore_barrier()
    def relu_chunk(buf, row0, nrows):
        pltpu.sync_copy(acc.at[pl.ds(row0, nrows)], buf.at[pl.ds(0, nrows)])

        def relu_row(r, carry):
            for k in range(HALF // 16):
                cols = pl.ds(k * 16, 16)
                buf[r, cols] = jnp.maximum(buf[r, cols], 0.0)
            return carry

        lax.fori_loop(0, nrows, relu_row, 0)
        pltpu.sync_copy(buf.at[pl.ds(0, nrows)],
                        ht_hbm.at[c].at[pl.ds(row0, nrows)])

    for j in range(7):
        relu_chunk(rows_a if j % 2 == 0 else rows_b,
                   stripe + j * CHUNK, CHUNK)
    relu_chunk(rows_b, stripe + 7 * CHUNK, ROWS_PER_TILE - 7 * CHUNK)


def _gather_body(ht_hbm, src_hbm, dst_hbm, hsrc_hbm, hdst_hbm,
                 sidx, didx, srows_a, srows_b, drows_a, drows_b, table,
                 g_sa, g_sb, g_da, g_db, st_sa, st_sb, st_da, st_db):
    c = lax.axis_index("c")
    s = lax.axis_index("s")
    base = s * E_PER_S
    col = c * HALF
    stripe = s * ROWS_PER_TILE
    pltpu.sync_copy(ht_hbm.at[c].at[pl.ds(stripe, ROWS_PER_TILE)],
                    table.at[pl.ds(stripe, ROWS_PER_TILE)])
    pltpu.sync_copy(src_hbm.at[s], sidx)
    pltpu.sync_copy(dst_hbm.at[s], didx)
    plsc.subcore_barrier()
    pltpu.async_copy(table.at[sidx.at[0]], srows_a, g_sa)
    pltpu.async_copy(table.at[didx.at[0]], drows_a, g_da)

    def pair(i, carry):
        a, b = 2 * i, 2 * i + 1
        off_a = base + a * CHUNK
        off_b = base + b * CHUNK

        @pl.when(i >= 1)
        def _():
            pltpu.make_async_copy(
                srows_b, hsrc_hbm.at[pl.ds(off_b, CHUNK), pl.ds(col, HALF)],
                st_sb).wait()
            pltpu.make_async_copy(
                drows_b, hdst_hbm.at[pl.ds(off_b, CHUNK), pl.ds(col, HALF)],
                st_db).wait()

        gsb = pltpu.async_copy(table.at[sidx.at[b]], srows_b, g_sb)
        gdb = pltpu.async_copy(table.at[didx.at[b]], drows_b, g_db)

        pltpu.make_async_copy(table.at[sidx.at[a]], srows_a, g_sa).wait()
        ssa = pltpu.async_copy(
            srows_a, hsrc_hbm.at[pl.ds(off_a, CHUNK), pl.ds(col, HALF)], st_sa)
        pltpu.make_async_copy(table.at[didx.at[a]], drows_a, g_da).wait()
        sda = pltpu.async_copy(
            drows_a, hdst_hbm.at[pl.ds(off_a, CHUNK), pl.ds(col, HALF)], st_da)

        ssa.wait()

        @pl.when(i < N_PAIR_S - 1)
        def _():
            pltpu.async_copy(table.at[sidx.at[a + 2]], srows_a, g_sa)

        sda.wait()

        @pl.when(i < N_PAIR_S - 1)
        def _():
            pltpu.async_copy(table.at[didx.at[a + 2]], drows_a, g_da)

        gsb.wait()
        pltpu.async_copy(srows_b, hsrc_hbm.at[pl.ds(off_b, CHUNK),
                                              pl.ds(col, HALF)], st_sb)
        gdb.wait()
        pltpu.async_copy(drows_b, hdst_hbm.at[pl.ds(off_b, CHUNK),
                                              pl.ds(col, HALF)], st_db)
        return carry

    lax.fori_loop(0, N_PAIR_S, pair, 0)
    off_b = base + (N_CHUNK_S - 1) * CHUNK
    pltpu.make_async_copy(
        srows_b, hsrc_hbm.at[pl.ds(off_b, CHUNK), pl.ds(col, HALF)],
        st_sb).wait()
    pltpu.make_async_copy(
        drows_b, hdst_hbm.at[pl.ds(off_b, CHUNK), pl.ds(col, HALF)],
        st_db).wait()


@functools.lru_cache(maxsize=1)
def _sc_kernels():
    mesh = plsc.VectorSubcoreMesh(
        core_axis_name="c", subcore_axis_name="s",
        num_cores=NC, num_subcores=NS)
    seg = functools.partial(
        pl.kernel,
        out_type=jax.ShapeDtypeStruct((NC, N_NODES, HALF), jnp.float32),
        mesh=mesh,
        compiler_params=pltpu.CompilerParams(use_tc_tiling_on_sc=False),
        scratch_types=[
            pltpu.VMEM((N_CHUNK_S, CHUNK), jnp.int32),
            pltpu.VMEM((N_CHUNK_S, CHUNK), jnp.int32),
            pltpu.VMEM((CHUNK, HALF), jnp.float32),
            pltpu.VMEM((CHUNK, HALF), jnp.float32),
            pltpu.VMEM_SHARED((ACC_ROWS, HALF), jnp.float32),
            pltpu.VMEM_SHARED((ACC_ROWS, HALF), jnp.float32),
            pltpu.SemaphoreType.DMA,
            pltpu.SemaphoreType.DMA,
            pltpu.SemaphoreType.DMA,
            pltpu.SemaphoreType.DMA,
        ],
    )(_seg_body)
    gather = functools.partial(
        pl.kernel,
        out_type=[
            jax.ShapeDtypeStruct((E_TOTAL, DIM), jnp.float32),
            jax.ShapeDtypeStruct((E_TOTAL, DIM), jnp.float32),
        ],
        mesh=mesh,
        compiler_params=pltpu.CompilerParams(use_tc_tiling_on_sc=False),
        scratch_types=[
            pltpu.VMEM((N_CHUNK_S, CHUNK), jnp.int32),
            pltpu.VMEM((N_CHUNK_S, CHUNK), jnp.int32),
            pltpu.VMEM((CHUNK, HALF), jnp.float32),
            pltpu.VMEM((CHUNK, HALF), jnp.float32),
            pltpu.VMEM((CHUNK, HALF), jnp.float32),
            pltpu.VMEM((CHUNK, HALF), jnp.float32),
            pltpu.VMEM_SHARED((N_NODES, HALF), jnp.float32),
            pltpu.SemaphoreType.DMA,
            pltpu.SemaphoreType.DMA,
            pltpu.SemaphoreType.DMA,
            pltpu.SemaphoreType.DMA,
            pltpu.SemaphoreType.DMA,
            pltpu.SemaphoreType.DMA,
            pltpu.SemaphoreType.DMA,
            pltpu.SemaphoreType.DMA,
        ],
    )(_gather_body)
    return seg, gather


_ROW_BLK = 1000
_N_BLK = N_NODES // _ROW_BLK


def _relu_body(pre_ref, ht_ref):
    ht_ref[...] = jnp.maximum(pre_ref[...], 0.0)


_relu_call = pl.pallas_call(
    _relu_body,
    grid=(_N_BLK,),
    in_specs=[pl.BlockSpec((NC, _ROW_BLK, HALF), lambda i: (0, i, 0))],
    out_specs=pl.BlockSpec((NC, _ROW_BLK, HALF), lambda i: (0, i, 0)),
    out_shape=jax.ShapeDtypeStruct((NC, N_NODES, HALF), jnp.float32),
)

_mm_call = pl.pallas_call(
    _mm_body,
    grid=(_N_BLK,),
    in_specs=[
        pl.BlockSpec((_ROW_BLK, DIM), lambda i: (i, 0)),
        pl.BlockSpec((_ROW_BLK, DIM), lambda i: (i, 0)),
        pl.BlockSpec((DIM, DIM), lambda i: (0, 0)),
        pl.BlockSpec((DIM, DIM), lambda i: (0, 0)),
        pl.BlockSpec((1, DIM), lambda i: (0, 0)),
        pl.BlockSpec((DIM, DIM), lambda i: (0, 0)),
        pl.BlockSpec((DIM, DIM), lambda i: (0, 0)),
    ],
    out_specs=[
        pl.BlockSpec((NC, _ROW_BLK, HALF), lambda i: (0, i, 0)),
        pl.BlockSpec((NC, _ROW_BLK, HALF), lambda i: (0, i, 0)),
    ],
    out_shape=[
        jax.ShapeDtypeStruct((NC, N_NODES, HALF), jnp.float32),
        jax.ShapeDtypeStruct((NC, N_NODES, HALF), jnp.float32),
    ],
)


def kernel(edge_index, t, msg, x_src, x_dst, W_src, b_src, W_dst, b_dst,
           W_self, W_msg):
    del t, msg
    src3s = edge_index[0].reshape(NS, N_CHUNK_S, CHUNK)
    dst3s = edge_index[1].reshape(NS, N_CHUNK_S, CHUNK)
    bias = (b_src + b_dst).reshape(1, DIM)
    seg_kernel, gather_kernel = _sc_kernels()
    hst, hwt = _mm_call(x_src, x_dst, W_src, W_dst, bias, W_self, W_msg)
    ht = seg_kernel(hwt, hst, src3s, dst3s)
    h_src, h_dst = gather_kernel(ht, src3s, dst3s)
    return (h_src, h_dst)


# speedup vs baseline: 1.0727x; 1.0034x over previous
import functools

import jax
import jax.numpy as jnp
from jax import lax
from jax.experimental import pallas as pl
from jax.experimental.pallas import tpu as pltpu
from jax.experimental.pallas import tpu_sc as plsc

N_NODES = 10000
E_TOTAL = 320000
DIM = 128
HALF = DIM // 2

NC = 2
NS = 16
NW = NC * NS
E_PER_S = E_TOTAL // NS
CHUNK = 80
N_CHUNK_S = E_PER_S // CHUNK
N_PAIR_S = N_CHUNK_S // 2
CHUNK_G = 125
N_CHUNK_G = E_PER_S // CHUNK_G
N_PAIR_G = N_CHUNK_G // 2
ROWS_PER_TILE = N_NODES // NS
ACC_ROWS = N_NODES


def _mm_body(xs_ref, xd_ref, ws_ref, wd_ref, b_ref, wself_ref, wmsg_ref,
             hst_ref, hwt_ref):
    hp = (jnp.dot(xs_ref[...], ws_ref[...], preferred_element_type=jnp.float32)
          + jnp.dot(xd_ref[...], wd_ref[...], preferred_element_type=jnp.float32)
          + b_ref[...])
    hs = jnp.dot(hp, wself_ref[...], preferred_element_type=jnp.float32)
    hst_ref[...] = jnp.stack([hs[:, :HALF], hs[:, HALF:]], axis=0)
    hw = jnp.dot(hp, wmsg_ref[...], preferred_element_type=jnp.float32)
    hwt_ref[...] = jnp.stack([hw[:, :HALF], hw[:, HALF:]], axis=0)


def _seg_body(hwt_hbm, hst_hbm, src_hbm, dst_hbm, ht_hbm,
              sidx, didx, rows_a, rows_b, table, acc,
              g_a, g_b, sc_a, sc_b):
    c = lax.axis_index("c")
    s = lax.axis_index("s")
    stripe = s * ROWS_PER_TILE
    pltpu.sync_copy(hst_hbm.at[c].at[pl.ds(stripe, ROWS_PER_TILE)],
                    acc.at[pl.ds(stripe, ROWS_PER_TILE)])
    pltpu.sync_copy(hwt_hbm.at[c].at[pl.ds(stripe, ROWS_PER_TILE)],
                    table.at[pl.ds(stripe, ROWS_PER_TILE)])
    pltpu.sync_copy(src_hbm.at[s], sidx)
    pltpu.sync_copy(dst_hbm.at[s], didx)
    plsc.subcore_barrier()

    pltpu.async_copy(table.at[sidx.at[0]], rows_a, g_a)

    def pair(i, carry):
        a, b = 2 * i, 2 * i + 1

        @pl.when(i >= 1)
        def _():
            pltpu.make_async_copy(rows_b, acc.at[didx.at[0]], sc_b).wait()

        gb = pltpu.async_copy(table.at[sidx.at[b]], rows_b, g_b)
        pltpu.make_async_copy(table.at[sidx.at[a]], rows_a, g_a).wait()
        sca = pltpu.async_copy(rows_a, acc.at[didx.at[a]], sc_a, add=True)
        sca.wait()

        @pl.when(i < N_PAIR_S - 1)
        def _():
            pltpu.async_copy(table.at[sidx.at[a + 2]], rows_a, g_a)

        gb.wait()
        pltpu.async_copy(rows_b, acc.at[didx.at[b]], sc_b, add=True)
        return carry

    lax.fori_loop(0, N_PAIR_S, pair, 0)
    pltpu.make_async_copy(rows_b, acc.at[didx.at[0]], sc_b).wait()

    plsc.subcore_barrier()
    def relu_chunk(buf, row0, nrows):
        pltpu.sync_copy(acc.at[pl.ds(row0, nrows)], buf.at[pl.ds(0, nrows)])

        def relu_row(r, carry):
            for k in range(HALF // 16):
                cols = pl.ds(k * 16, 16)
                buf[r, cols] = jnp.maximum(buf[r, cols], 0.0)
            return carry

        lax.fori_loop(0, nrows, relu_row, 0)
        pltpu.sync_copy(buf.at[pl.ds(0, nrows)],
                        ht_hbm.at[c].at[pl.ds(row0, nrows)])

    for j in range(7):
        relu_chunk(rows_a if j % 2 == 0 else rows_b,
                   stripe + j * CHUNK, CHUNK)
    relu_chunk(rows_b, stripe + 7 * CHUNK, ROWS_PER_TILE - 7 * CHUNK)


def _gather_body(ht_hbm, src_hbm, dst_hbm, hsrc_hbm, hdst_hbm,
                 sidx, didx, srows_a, srows_b, drows_a, drows_b, table,
                 g_sa, g_sb, g_da, g_db, st_sa, st_sb, st_da, st_db):
    c = lax.axis_index("c")
    s = lax.axis_index("s")
    base = s * E_PER_S
    col = c * HALF
    stripe = s * ROWS_PER_TILE
    pltpu.sync_copy(ht_hbm.at[c].at[pl.ds(stripe, ROWS_PER_TILE)],
                    table.at[pl.ds(stripe, ROWS_PER_TILE)])
    pltpu.sync_copy(src_hbm.at[s], sidx)
    pltpu.sync_copy(dst_hbm.at[s], didx)
    plsc.subcore_barrier()
    pltpu.async_copy(table.at[sidx.at[0]], srows_a, g_sa)
    pltpu.async_copy(table.at[didx.at[0]], drows_a, g_da)

    def pair(i, carry):
        a, b = 2 * i, 2 * i + 1
        off_a = base + a * CHUNK_G
        off_b = base + b * CHUNK_G

        @pl.when(i >= 1)
        def _():
            pltpu.make_async_copy(
                srows_b, hsrc_hbm.at[pl.ds(off_b, CHUNK_G), pl.ds(col, HALF)],
                st_sb).wait()
            pltpu.make_async_copy(
                drows_b, hdst_hbm.at[pl.ds(off_b, CHUNK_G), pl.ds(col, HALF)],
                st_db).wait()

        gsb = pltpu.async_copy(table.at[sidx.at[b]], srows_b, g_sb)
        gdb = pltpu.async_copy(table.at[didx.at[b]], drows_b, g_db)

        pltpu.make_async_copy(table.at[sidx.at[a]], srows_a, g_sa).wait()
        ssa = pltpu.async_copy(
            srows_a, hsrc_hbm.at[pl.ds(off_a, CHUNK_G), pl.ds(col, HALF)], st_sa)
        pltpu.make_async_copy(table.at[didx.at[a]], drows_a, g_da).wait()
        sda = pltpu.async_copy(
            drows_a, hdst_hbm.at[pl.ds(off_a, CHUNK_G), pl.ds(col, HALF)], st_da)

        ssa.wait()

        @pl.when(i < N_PAIR_G - 1)
        def _():
            pltpu.async_copy(table.at[sidx.at[a + 2]], srows_a, g_sa)

        sda.wait()

        @pl.when(i < N_PAIR_G - 1)
        def _():
            pltpu.async_copy(table.at[didx.at[a + 2]], drows_a, g_da)

        gsb.wait()
        pltpu.async_copy(srows_b, hsrc_hbm.at[pl.ds(off_b, CHUNK_G),
                                              pl.ds(col, HALF)], st_sb)
        gdb.wait()
        pltpu.async_copy(drows_b, hdst_hbm.at[pl.ds(off_b, CHUNK_G),
                                              pl.ds(col, HALF)], st_db)
        return carry

    lax.fori_loop(0, N_PAIR_G, pair, 0)
    off_b = base + (N_CHUNK_G - 1) * CHUNK_G
    pltpu.make_async_copy(
        srows_b, hsrc_hbm.at[pl.ds(off_b, CHUNK_G), pl.ds(col, HALF)],
        st_sb).wait()
    pltpu.make_async_copy(
        drows_b, hdst_hbm.at[pl.ds(off_b, CHUNK_G), pl.ds(col, HALF)],
        st_db).wait()


@functools.lru_cache(maxsize=1)
def _sc_kernels():
    mesh = plsc.VectorSubcoreMesh(
        core_axis_name="c", subcore_axis_name="s",
        num_cores=NC, num_subcores=NS)
    seg = functools.partial(
        pl.kernel,
        out_type=jax.ShapeDtypeStruct((NC, N_NODES, HALF), jnp.float32),
        mesh=mesh,
        compiler_params=pltpu.CompilerParams(use_tc_tiling_on_sc=False),
        scratch_types=[
            pltpu.VMEM((N_CHUNK_S, CHUNK), jnp.int32),
            pltpu.VMEM((N_CHUNK_S, CHUNK), jnp.int32),
            pltpu.VMEM((CHUNK, HALF), jnp.float32),
            pltpu.VMEM((CHUNK, HALF), jnp.float32),
            pltpu.VMEM_SHARED((ACC_ROWS, HALF), jnp.float32),
            pltpu.VMEM_SHARED((ACC_ROWS, HALF), jnp.float32),
            pltpu.SemaphoreType.DMA,
            pltpu.SemaphoreType.DMA,
            pltpu.SemaphoreType.DMA,
            pltpu.SemaphoreType.DMA,
        ],
    )(_seg_body)
    gather = functools.partial(
        pl.kernel,
        out_type=[
            jax.ShapeDtypeStruct((E_TOTAL, DIM), jnp.float32),
            jax.ShapeDtypeStruct((E_TOTAL, DIM), jnp.float32),
        ],
        mesh=mesh,
        compiler_params=pltpu.CompilerParams(use_tc_tiling_on_sc=False),
        scratch_types=[
            pltpu.VMEM((N_CHUNK_G, CHUNK_G), jnp.int32),
            pltpu.VMEM((N_CHUNK_G, CHUNK_G), jnp.int32),
            pltpu.VMEM((CHUNK_G, HALF), jnp.float32),
            pltpu.VMEM((CHUNK_G, HALF), jnp.float32),
            pltpu.VMEM((CHUNK_G, HALF), jnp.float32),
            pltpu.VMEM((CHUNK_G, HALF), jnp.float32),
            pltpu.VMEM_SHARED((N_NODES, HALF), jnp.float32),
            pltpu.SemaphoreType.DMA,
            pltpu.SemaphoreType.DMA,
            pltpu.SemaphoreType.DMA,
            pltpu.SemaphoreType.DMA,
            pltpu.SemaphoreType.DMA,
            pltpu.SemaphoreType.DMA,
            pltpu.SemaphoreType.DMA,
            pltpu.SemaphoreType.DMA,
        ],
    )(_gather_body)
    return seg, gather


_ROW_BLK = 1000
_N_BLK = N_NODES // _ROW_BLK


def _relu_body(pre_ref, ht_ref):
    ht_ref[...] = jnp.maximum(pre_ref[...], 0.0)


_relu_call = pl.pallas_call(
    _relu_body,
    grid=(_N_BLK,),
    in_specs=[pl.BlockSpec((NC, _ROW_BLK, HALF), lambda i: (0, i, 0))],
    out_specs=pl.BlockSpec((NC, _ROW_BLK, HALF), lambda i: (0, i, 0)),
    out_shape=jax.ShapeDtypeStruct((NC, N_NODES, HALF), jnp.float32),
)

_mm_call = pl.pallas_call(
    _mm_body,
    grid=(_N_BLK,),
    in_specs=[
        pl.BlockSpec((_ROW_BLK, DIM), lambda i: (i, 0)),
        pl.BlockSpec((_ROW_BLK, DIM), lambda i: (i, 0)),
        pl.BlockSpec((DIM, DIM), lambda i: (0, 0)),
        pl.BlockSpec((DIM, DIM), lambda i: (0, 0)),
        pl.BlockSpec((1, DIM), lambda i: (0, 0)),
        pl.BlockSpec((DIM, DIM), lambda i: (0, 0)),
        pl.BlockSpec((DIM, DIM), lambda i: (0, 0)),
    ],
    out_specs=[
        pl.BlockSpec((NC, _ROW_BLK, HALF), lambda i: (0, i, 0)),
        pl.BlockSpec((NC, _ROW_BLK, HALF), lambda i: (0, i, 0)),
    ],
    out_shape=[
        jax.ShapeDtypeStruct((NC, N_NODES, HALF), jnp.float32),
        jax.ShapeDtypeStruct((NC, N_NODES, HALF), jnp.float32),
    ],
)


def kernel(edge_index, t, msg, x_src, x_dst, W_src, b_src, W_dst, b_dst,
           W_self, W_msg):
    del t, msg
    src3s = edge_index[0].reshape(NS, N_CHUNK_S, CHUNK)
    dst3s = edge_index[1].reshape(NS, N_CHUNK_S, CHUNK)
    src3g = edge_index[0].reshape(NS, N_CHUNK_G, CHUNK_G)
    dst3g = edge_index[1].reshape(NS, N_CHUNK_G, CHUNK_G)
    bias = (b_src + b_dst).reshape(1, DIM)
    seg_kernel, gather_kernel = _sc_kernels()
    hst, hwt = _mm_call(x_src, x_dst, W_src, W_dst, bias, W_self, W_msg)
    ht = seg_kernel(hwt, hst, src3s, dst3s)
    h_src, h_dst = gather_kernel(ht, src3g, dst3g)
    return (h_src, h_dst)
